# baseline (speedup 1.0000x reference)
import jax
import jax.numpy as jnp
from jax import lax
from jax.experimental import pallas as pl
from jax.experimental.pallas import tpu as pltpu
from jax.experimental.pallas import tpu_sc as plsc

NUM_TIMES = 288
NUM_WEEKDAYS = 7
TIME_DIM = 64
WEEKDAY_DIM = 64
MODEL_DIM = 512
HALF_COLS = MODEL_DIM // 2

NC = 2
NS = 16
NW = NC * NS
L = 16

TOKENS = 16 * 12 * 512
ROWS_PER_W = TOKENS // NW
CHUNK = 32
NBUF = 2
NCHUNK = ROWS_PER_W // CHUNK
COL_UNROLL = 8


def _tables_body(time_ref, wd_ref, tt_ref, wt_ref, wdt_ref, ww_ref,
                 pt_ref, pw_ref, t_ref, w_ref):
    p_time = jnp.dot(tt_ref[...], wt_ref[...],
                     preferred_element_type=jnp.float32)
    p_wd = jnp.dot(wdt_ref[...], ww_ref[...],
                   preferred_element_type=jnp.float32)
    pt_bf = p_time.astype(jnp.bfloat16)
    pw_bf = p_wd.astype(jnp.bfloat16)
    pt_ref[...] = jnp.stack(
        [pt_bf[:, :HALF_COLS], pt_bf[:, HALF_COLS:]], axis=-1)
    pw_ref[...] = jnp.stack(
        [pw_bf[:, :HALF_COLS], pw_bf[:, HALF_COLS:]], axis=-1)
    t_ref[...] = time_ref[...] * HALF_COLS
    w_ref[...] = jnp.broadcast_to(wd_ref[...] * HALF_COLS, w_ref.shape)


def _build_tables(time2d, wd2d, time_table, W_time, weekday_table, W_weekday):
    n_bt = time2d.shape[0]
    pt_bf, pw_bf, t256, w256 = pl.pallas_call(
        _tables_body,
        out_shape=[
            jax.ShapeDtypeStruct((NUM_TIMES, HALF_COLS, 2), jnp.bfloat16),
            jax.ShapeDtypeStruct((NUM_WEEKDAYS, HALF_COLS, 2), jnp.bfloat16),
            jax.ShapeDtypeStruct((n_bt, TOKENS // n_bt), jnp.int32),
            jax.ShapeDtypeStruct((n_bt, TOKENS // n_bt), jnp.int32),
        ],
    )(time2d, wd2d, time_table, W_time, weekday_table, W_weekday)
    pt_i32 = lax.bitcast_convert_type(pt_bf, jnp.int32).reshape(-1)
    pw_i32 = lax.bitcast_convert_type(pw_bf, jnp.int32).reshape(-1)
    return pt_i32, pw_i32, t256.reshape(-1), w256.reshape(-1)


def _gather_body(pt_hbm, pw_hbm, t_hbm, w_hbm, out_hbm,
                 pt_v, pw_v, t_v, w_v, fbuf0, fbuf1, wsems):
    fbufs = [fbuf0, fbuf1]
    wid = lax.axis_index("s") * NC + lax.axis_index("c")
    base = wid * ROWS_PER_W
    pltpu.sync_copy(pt_hbm, pt_v)
    pltpu.sync_copy(pw_hbm, pw_v)
    pltpu.sync_copy(t_hbm.at[pl.ds(base, ROWS_PER_W)], t_v)
    pltpu.sync_copy(w_hbm.at[pl.ds(base, ROWS_PER_W)], w_v)

    lanes = jnp.arange(L, dtype=jnp.int32)

    def compute_chunk(c, b):
        fbuf = fbufs[b]
        for g in range(CHUNK // L):
            toff = c * CHUNK + g * L
            t_vec = t_v[pl.ds(toff, L)]
            w_vec = w_v[pl.ds(toff, L)]
            pos_e = lanes * MODEL_DIM + (g * L * MODEL_DIM)
            pos_o = pos_e + HALF_COLS

            def cols(k0):
                for u in range(COL_UNROLL):
                    k = k0 + u
                    pt_pair = plsc.load_gather(pt_v, [t_vec + k])
                    pw_pair = plsc.load_gather(pw_v, [w_vec + k])
                    a = plsc.unpack(plsc.bitcast(pt_pair, jnp.bfloat16),
                                    format=plsc.PackFormat.INTERLEAVED)
                    bb = plsc.unpack(plsc.bitcast(pw_pair, jnp.bfloat16),
                                     format=plsc.PackFormat.INTERLEAVED)
                    plsc.store_scatter(fbuf, [pos_e + k], a[0] + bb[0])
                    plsc.store_scatter(fbuf, [pos_o + k], a[1] + bb[1])

            pl.loop(0, HALF_COLS, step=COL_UNROLL)(cols)

    def write(c, b):
        pltpu.async_copy(
            fbufs[b],
            out_hbm.at[pl.ds((base + c * CHUNK) * MODEL_DIM,
                             CHUNK * MODEL_DIM)],
            wsems.at[b])

    def wait_write(b):
        pltpu.make_async_copy(
            fbufs[b],
            out_hbm.at[pl.ds(0, CHUNK * MODEL_DIM)],
            wsems.at[b]).wait()

    def body(g):
        for b in range(NBUF):
            c = g + b

            @pl.when(c >= NBUF)
            def _():
                wait_write(b)

            compute_chunk(c, b)
            write(c, b)

    pl.loop(0, NCHUNK, step=NBUF)(body)
    for b in range(NBUF):
        wait_write(b)


def _sc_gather(pt_i32, pw_i32, tflat, wflat):
    mesh = plsc.VectorSubcoreMesh(core_axis_name="c", subcore_axis_name="s")
    run = pl.kernel(
        _gather_body,
        out_type=jax.ShapeDtypeStruct((TOKENS * MODEL_DIM,), jnp.float32),
        mesh=mesh,
        compiler_params=pltpu.CompilerParams(needs_layout_passes=False),
        scratch_types=[
            pltpu.VMEM((NUM_TIMES * HALF_COLS,), jnp.int32),
            pltpu.VMEM((NUM_WEEKDAYS * HALF_COLS,), jnp.int32),
            pltpu.VMEM((ROWS_PER_W,), jnp.int32),
            pltpu.VMEM((ROWS_PER_W,), jnp.int32),
            pltpu.VMEM((CHUNK * MODEL_DIM,), jnp.float32),
            pltpu.VMEM((CHUNK * MODEL_DIM,), jnp.float32),
            pltpu.SemaphoreType.DMA((NBUF,)),
        ],
    )
    return run(pt_i32, pw_i32, tflat, wflat)


@jax.jit
def kernel(time, weekday, time_table, W_time, weekday_table, W_weekday):
    B, T, N = time.shape
    time2d = time.reshape(B * T, N).astype(jnp.int32)
    wd2d = weekday.reshape(B * T, 1).astype(jnp.int32)
    pt_i32, pw_i32, tflat, wflat = _build_tables(
        time2d, wd2d, time_table, W_time, weekday_table, W_weekday)
    out = _sc_gather(pt_i32, pw_i32, tflat, wflat)
    return out.reshape(B, T, N, MODEL_DIM)


# speedup vs baseline: 1.6645x; 1.6645x over previous
import jax
import jax.numpy as jnp
from jax import lax
from jax.experimental import pallas as pl
from jax.experimental.pallas import tpu as pltpu
from jax.experimental.pallas import tpu_sc as plsc

NUM_TIMES = 288
NUM_WEEKDAYS = 7
TIME_DIM = 64
WEEKDAY_DIM = 64
MODEL_DIM = 512
HALF_COLS = MODEL_DIM // 2

NC = 2
NS = 16
NW = NC * NS
L = 16

TOKENS = 16 * 12 * 512
ROWS_PER_W = TOKENS // NW
CHUNK = 32
NBUF = 2
NCHUNK = ROWS_PER_W // CHUNK
COL_UNROLL = 8


def _tables_body(time_ref, wd_ref, tt_ref, wt_ref, wdt_ref, ww_ref,
                 pt_ref, pw_ref, t_ref, w_ref):
    p_time = jnp.dot(tt_ref[...], wt_ref[...],
                     preferred_element_type=jnp.float32)
    p_wd = jnp.dot(wdt_ref[...], ww_ref[...],
                   preferred_element_type=jnp.float32)
    pt_bf = p_time.astype(jnp.bfloat16)
    pw_bf = p_wd.astype(jnp.bfloat16)
    pt_ref[...] = jnp.stack(
        [pt_bf[:, :HALF_COLS], pt_bf[:, HALF_COLS:]], axis=-1)
    pw_ref[...] = jnp.stack(
        [pw_bf[:, :HALF_COLS], pw_bf[:, HALF_COLS:]], axis=-1)
    t_ref[...] = time_ref[...] * HALF_COLS
    w_ref[...] = jnp.broadcast_to(wd_ref[...] * HALF_COLS, w_ref.shape)


def _build_tables(time2d, wd2d, time_table, W_time, weekday_table, W_weekday):
    n_bt = time2d.shape[0]
    pt_bf, pw_bf, t256, w256 = pl.pallas_call(
        _tables_body,
        out_shape=[
            jax.ShapeDtypeStruct((NUM_TIMES, HALF_COLS, 2), jnp.bfloat16),
            jax.ShapeDtypeStruct((NUM_WEEKDAYS, HALF_COLS, 2), jnp.bfloat16),
            jax.ShapeDtypeStruct((n_bt, TOKENS // n_bt), jnp.int32),
            jax.ShapeDtypeStruct((n_bt, TOKENS // n_bt), jnp.int32),
        ],
    )(time2d, wd2d, time_table, W_time, weekday_table, W_weekday)
    pt_i32 = lax.bitcast_convert_type(pt_bf, jnp.int32).reshape(-1)
    pw_i32 = lax.bitcast_convert_type(pw_bf, jnp.int32).reshape(-1)
    return pt_i32, pw_i32, t256.reshape(-1), w256.reshape(-1)


def _gather_body(pt_hbm, pw_hbm, t_hbm, w_hbm, out_hbm,
                 pt_v, pw_v, t_v, w_v, fbuf0, fbuf1, wsems):
    fbufs = [fbuf0, fbuf1]
    wid = lax.axis_index("s") * NC + lax.axis_index("c")
    base = wid * ROWS_PER_W
    pltpu.sync_copy(pt_hbm, pt_v)
    pltpu.sync_copy(pw_hbm, pw_v)
    pltpu.sync_copy(t_hbm.at[pl.ds(base, ROWS_PER_W)], t_v)
    pltpu.sync_copy(w_hbm.at[pl.ds(base, ROWS_PER_W)], w_v)

    lanes = jnp.arange(L, dtype=jnp.int32)

    def compute_chunk(c, b):
        fbuf = fbufs[b]
        for g in range(CHUNK // L):
            toff = c * CHUNK + g * L
            t_vec = t_v[pl.ds(toff, L)]
            w_vec = w_v[pl.ds(toff, L)]
            pos_e = lanes * MODEL_DIM + (g * L * MODEL_DIM)
            pos_o = pos_e + HALF_COLS

            def cols(k):
                pt_pair = plsc.load_gather(pt_v, [t_vec + k])
                pw_pair = plsc.load_gather(pw_v, [w_vec + k])
                a = plsc.unpack(plsc.bitcast(pt_pair, jnp.bfloat16),
                                format=plsc.PackFormat.INTERLEAVED)
                bb = plsc.unpack(plsc.bitcast(pw_pair, jnp.bfloat16),
                                 format=plsc.PackFormat.INTERLEAVED)
                plsc.store_scatter(fbuf, [pos_e + k], a[0] + bb[0])
                plsc.store_scatter(fbuf, [pos_o + k], a[1] + bb[1])

            plsc.parallel_loop(0, HALF_COLS, unroll=COL_UNROLL)(cols)

    def write(c, b):
        pltpu.async_copy(
            fbufs[b],
            out_hbm.at[pl.ds((base + c * CHUNK) * MODEL_DIM,
                             CHUNK * MODEL_DIM)],
            wsems.at[b])

    def wait_write(b):
        pltpu.make_async_copy(
            fbufs[b],
            out_hbm.at[pl.ds(0, CHUNK * MODEL_DIM)],
            wsems.at[b]).wait()

    def body(g):
        for b in range(NBUF):
            c = g + b

            @pl.when(c >= NBUF)
            def _():
                wait_write(b)

            compute_chunk(c, b)
            write(c, b)

    pl.loop(0, NCHUNK, step=NBUF)(body)
    for b in range(NBUF):
        wait_write(b)


def _sc_gather(pt_i32, pw_i32, tflat, wflat):
    mesh = plsc.VectorSubcoreMesh(core_axis_name="c", subcore_axis_name="s")
    run = pl.kernel(
        _gather_body,
        out_type=jax.ShapeDtypeStruct((TOKENS * MODEL_DIM,), jnp.float32),
        mesh=mesh,
        compiler_params=pltpu.CompilerParams(needs_layout_passes=False),
        scratch_types=[
            pltpu.VMEM((NUM_TIMES * HALF_COLS,), jnp.int32),
            pltpu.VMEM((NUM_WEEKDAYS * HALF_COLS,), jnp.int32),
            pltpu.VMEM((ROWS_PER_W,), jnp.int32),
            pltpu.VMEM((ROWS_PER_W,), jnp.int32),
            pltpu.VMEM((CHUNK * MODEL_DIM,), jnp.float32),
            pltpu.VMEM((CHUNK * MODEL_DIM,), jnp.float32),
            pltpu.SemaphoreType.DMA((NBUF,)),
        ],
    )
    return run(pt_i32, pw_i32, tflat, wflat)


@jax.jit
def kernel(time, weekday, time_table, W_time, weekday_table, W_weekday):
    B, T, N = time.shape
    time2d = time.reshape(B * T, N).astype(jnp.int32)
    wd2d = weekday.reshape(B * T, 1).astype(jnp.int32)
    pt_i32, pw_i32, tflat, wflat = _build_tables(
        time2d, wd2d, time_table, W_time, weekday_table, W_weekday)
    out = _sc_gather(pt_i32, pw_i32, tflat, wflat)
    return out.reshape(B, T, N, MODEL_DIM)


# speedup vs baseline: 5.1335x; 3.0841x over previous
import jax
import jax.numpy as jnp
from jax import lax
from jax.experimental import pallas as pl
from jax.experimental.pallas import tpu as pltpu
from jax.experimental.pallas import tpu_sc as plsc

NUM_TIMES = 288
NUM_WEEKDAYS = 7
TIME_DIM = 64
WEEKDAY_DIM = 64
MODEL_DIM = 512
HALF_COLS = MODEL_DIM // 2

NC = 2
NS = 16
NW = NC * NS
L = 16

TOKENS = 16 * 12 * 512
ROWS_PER_W = TOKENS // NW
CHUNK = 32
NBUF = 2
NCHUNK = ROWS_PER_W // CHUNK
COL_UNROLL = 8


def _tables_body(time_ref, wd_ref, tt_ref, wt_ref, wdt_ref, ww_ref,
                 pt_ref, pw_ref, t_ref, w_ref):
    p_time = jnp.dot(tt_ref[...], wt_ref[...],
                     preferred_element_type=jnp.float32)
    p_wd = jnp.dot(wdt_ref[...], ww_ref[...],
                   preferred_element_type=jnp.float32)
    pt_bf = p_time.astype(jnp.bfloat16)
    pw_bf = p_wd.astype(jnp.bfloat16)
    pt_ref[...] = jnp.stack(
        [pt_bf[:, :HALF_COLS], pt_bf[:, HALF_COLS:]], axis=-1)
    pw_ref[...] = jnp.stack(
        [pw_bf[:, :HALF_COLS], pw_bf[:, HALF_COLS:]], axis=-1)
    t_ref[...] = time_ref[...] * HALF_COLS
    w_ref[...] = jnp.broadcast_to(wd_ref[...] * HALF_COLS, w_ref.shape)


def _build_tables(time2d, wd2d, time_table, W_time, weekday_table, W_weekday):
    n_bt = time2d.shape[0]
    pt_bf, pw_bf, t256, w256 = pl.pallas_call(
        _tables_body,
        out_shape=[
            jax.ShapeDtypeStruct((NUM_TIMES, HALF_COLS, 2), jnp.bfloat16),
            jax.ShapeDtypeStruct((NUM_WEEKDAYS, HALF_COLS, 2), jnp.bfloat16),
            jax.ShapeDtypeStruct((n_bt, TOKENS // n_bt), jnp.int32),
            jax.ShapeDtypeStruct((n_bt, TOKENS // n_bt), jnp.int32),
        ],
    )(time2d, wd2d, time_table, W_time, weekday_table, W_weekday)
    pt_i32 = lax.bitcast_convert_type(pt_bf, jnp.int32).reshape(-1)
    pw_i32 = lax.bitcast_convert_type(pw_bf, jnp.int32).reshape(-1)
    return pt_i32, pw_i32, t256.reshape(-1), w256.reshape(-1)


def _gather_body(pt_hbm, pw_hbm, t_hbm, w_hbm, out_hbm,
                 pt_v, pw_v, t_v, w_v, fbuf0, fbuf1, wsems):
    fbufs = [fbuf0, fbuf1]
    wid = lax.axis_index("s") * NC + lax.axis_index("c")
    base = wid * ROWS_PER_W
    pltpu.sync_copy(pt_hbm, pt_v)
    pltpu.sync_copy(pw_hbm, pw_v)
    pltpu.sync_copy(t_hbm.at[pl.ds(base, ROWS_PER_W)], t_v)
    pltpu.sync_copy(w_hbm.at[pl.ds(base, ROWS_PER_W)], w_v)

    def compute_chunk(c, b):
        fbuf = fbufs[b]
        t_s = []
        for g in range(CHUNK // L):
            t_vec = t_v[pl.ds(c * CHUNK + g * L, L)]
            t_s.extend(t_vec[j] for j in range(L))
        w256 = w_v[pl.ds(c * CHUNK, L)][0]

        def kbody(k0):
            pw_pair = pw_v[pl.ds(w256 + k0, L)]
            bb = plsc.unpack(plsc.bitcast(pw_pair, jnp.bfloat16),
                             format=plsc.PackFormat.INTERLEAVED)
            for tok in range(CHUNK):
                pt_pair = pt_v[pl.ds(t_s[tok] + k0, L)]
                a = plsc.unpack(plsc.bitcast(pt_pair, jnp.bfloat16),
                                format=plsc.PackFormat.INTERLEAVED)
                outb = tok * MODEL_DIM
                fbuf[pl.ds(outb + k0, L)] = a[0] + bb[0]
                fbuf[pl.ds(outb + HALF_COLS + k0, L)] = a[1] + bb[1]

        plsc.parallel_loop(0, HALF_COLS, step=L)(kbody)

    def write(c, b):
        pltpu.async_copy(
            fbufs[b],
            out_hbm.at[pl.ds((base + c * CHUNK) * MODEL_DIM,
                             CHUNK * MODEL_DIM)],
            wsems.at[b])

    def wait_write(b):
        pltpu.make_async_copy(
            fbufs[b],
            out_hbm.at[pl.ds(0, CHUNK * MODEL_DIM)],
            wsems.at[b]).wait()

    def body(g):
        for b in range(NBUF):
            c = g + b

            @pl.when(c >= NBUF)
            def _():
                wait_write(b)

            compute_chunk(c, b)
            write(c, b)

    pl.loop(0, NCHUNK, step=NBUF)(body)
    for b in range(NBUF):
        wait_write(b)


def _sc_gather(pt_i32, pw_i32, tflat, wflat):
    mesh = plsc.VectorSubcoreMesh(core_axis_name="c", subcore_axis_name="s")
    run = pl.kernel(
        _gather_body,
        out_type=jax.ShapeDtypeStruct((TOKENS * MODEL_DIM,), jnp.float32),
        mesh=mesh,
        compiler_params=pltpu.CompilerParams(needs_layout_passes=False),
        scratch_types=[
            pltpu.VMEM((NUM_TIMES * HALF_COLS,), jnp.int32),
            pltpu.VMEM((NUM_WEEKDAYS * HALF_COLS,), jnp.int32),
            pltpu.VMEM((ROWS_PER_W,), jnp.int32),
            pltpu.VMEM((ROWS_PER_W,), jnp.int32),
            pltpu.VMEM((CHUNK * MODEL_DIM,), jnp.float32),
            pltpu.VMEM((CHUNK * MODEL_DIM,), jnp.float32),
            pltpu.SemaphoreType.DMA((NBUF,)),
        ],
    )
    return run(pt_i32, pw_i32, tflat, wflat)


@jax.jit
def kernel(time, weekday, time_table, W_time, weekday_table, W_weekday):
    B, T, N = time.shape
    time2d = time.reshape(B * T, N).astype(jnp.int32)
    wd2d = weekday.reshape(B * T, 1).astype(jnp.int32)
    pt_i32, pw_i32, tflat, wflat = _build_tables(
        time2d, wd2d, time_table, W_time, weekday_table, W_weekday)
    out = _sc_gather(pt_i32, pw_i32, tflat, wflat)
    return out.reshape(B, T, N, MODEL_DIM)


# speedup vs baseline: 5.2830x; 1.0291x over previous
import jax
import jax.numpy as jnp
from jax import lax
from jax.experimental import pallas as pl
from jax.experimental.pallas import tpu as pltpu
from jax.experimental.pallas import tpu_sc as plsc

NUM_TIMES = 288
NUM_WEEKDAYS = 7
TIME_DIM = 64
WEEKDAY_DIM = 64
MODEL_DIM = 512
HALF_COLS = MODEL_DIM // 2

NC = 2
NS = 16
NW = NC * NS
L = 16

TOKENS = 16 * 12 * 512
ROWS_PER_W = TOKENS // NW
CHUNK = 32
NBUF = 2
NCHUNK = ROWS_PER_W // CHUNK
COL_UNROLL = 8


def _tables_body(time_ref, wd_ref, tt_ref, wt_ref, wdt_ref, ww_ref,
                 pt_ref, pw_ref, t_ref, w_ref):
    p_time = jnp.dot(tt_ref[...], wt_ref[...],
                     preferred_element_type=jnp.float32)
    p_wd = jnp.dot(wdt_ref[...], ww_ref[...],
                   preferred_element_type=jnp.float32)
    pt_bf = p_time.astype(jnp.bfloat16)
    pw_bf = p_wd.astype(jnp.bfloat16)
    pt_ref[...] = jnp.stack(
        [pt_bf[:, :HALF_COLS], pt_bf[:, HALF_COLS:]], axis=-1)
    pw_ref[...] = jnp.stack(
        [pw_bf[:, :HALF_COLS], pw_bf[:, HALF_COLS:]], axis=-1)
    t_ref[...] = time_ref[...] * HALF_COLS
    w_ref[...] = jnp.broadcast_to(wd_ref[...] * HALF_COLS, w_ref.shape)


def _build_tables(time2d, wd2d, time_table, W_time, weekday_table, W_weekday):
    n_bt = time2d.shape[0]
    pt_bf, pw_bf, t256, w256 = pl.pallas_call(
        _tables_body,
        out_shape=[
            jax.ShapeDtypeStruct((NUM_TIMES, HALF_COLS, 2), jnp.bfloat16),
            jax.ShapeDtypeStruct((NUM_WEEKDAYS, HALF_COLS, 2), jnp.bfloat16),
            jax.ShapeDtypeStruct((n_bt, TOKENS // n_bt), jnp.int32),
            jax.ShapeDtypeStruct((n_bt, TOKENS // n_bt), jnp.int32),
        ],
    )(time2d, wd2d, time_table, W_time, weekday_table, W_weekday)
    pt_i32 = lax.bitcast_convert_type(pt_bf, jnp.int32).reshape(-1)
    pw_i32 = lax.bitcast_convert_type(pw_bf, jnp.int32).reshape(-1)
    return pt_i32, pw_i32, t256.reshape(-1), w256.reshape(-1)


def _gather_body(pt_hbm, pw_hbm, t_hbm, w_hbm, out_hbm,
                 pt_v, pw_v, t_v, w_v, fbuf0, fbuf1, wsems):
    fbufs = [fbuf0, fbuf1]
    wid = lax.axis_index("s") * NC + lax.axis_index("c")
    base = wid * ROWS_PER_W
    pltpu.sync_copy(pt_hbm, pt_v)
    pltpu.sync_copy(pw_hbm, pw_v)
    pltpu.sync_copy(t_hbm.at[pl.ds(base, ROWS_PER_W)], t_v)
    pltpu.sync_copy(w_hbm.at[pl.ds(base, ROWS_PER_W)], w_v)

    def compute_chunk(c, b):
        fbuf = fbufs[b]
        t_s = []
        for g in range(CHUNK // L):
            t_vec = t_v[pl.ds(c * CHUNK + g * L, L)]
            t_s.extend(t_vec[j] for j in range(L))
        w256 = w_v[pl.ds(c * CHUNK, L)][0]

        def kbody(k0):
            pw_pair = pw_v[pl.ds(w256 + k0, L)]
            bb = plsc.unpack(plsc.bitcast(pw_pair, jnp.bfloat16),
                             format=plsc.PackFormat.INTERLEAVED)
            for t0 in range(0, CHUNK, 8):
                pairs = [pt_v[pl.ds(t_s[t0 + j] + k0, L)] for j in range(8)]
                for j in range(8):
                    a = plsc.unpack(plsc.bitcast(pairs[j], jnp.bfloat16),
                                    format=plsc.PackFormat.INTERLEAVED)
                    outb = (t0 + j) * MODEL_DIM
                    fbuf[pl.ds(outb + k0, L)] = a[0] + bb[0]
                    fbuf[pl.ds(outb + HALF_COLS + k0, L)] = a[1] + bb[1]

        plsc.parallel_loop(0, HALF_COLS, step=L)(kbody)

    def write(c, b):
        pltpu.async_copy(
            fbufs[b],
            out_hbm.at[pl.ds((base + c * CHUNK) * MODEL_DIM,
                             CHUNK * MODEL_DIM)],
            wsems.at[b])

    def wait_write(b):
        pltpu.make_async_copy(
            fbufs[b],
            out_hbm.at[pl.ds(0, CHUNK * MODEL_DIM)],
            wsems.at[b]).wait()

    def body(g):
        for b in range(NBUF):
            c = g + b

            @pl.when(c >= NBUF)
            def _():
                wait_write(b)

            compute_chunk(c, b)
            write(c, b)

    pl.loop(0, NCHUNK, step=NBUF)(body)
    for b in range(NBUF):
        wait_write(b)


def _sc_gather(pt_i32, pw_i32, tflat, wflat):
    mesh = plsc.VectorSubcoreMesh(core_axis_name="c", subcore_axis_name="s")
    run = pl.kernel(
        _gather_body,
        out_type=jax.ShapeDtypeStruct((TOKENS * MODEL_DIM,), jnp.float32),
        mesh=mesh,
        compiler_params=pltpu.CompilerParams(needs_layout_passes=False),
        scratch_types=[
            pltpu.VMEM((NUM_TIMES * HALF_COLS,), jnp.int32),
            pltpu.VMEM((NUM_WEEKDAYS * HALF_COLS,), jnp.int32),
            pltpu.VMEM((ROWS_PER_W,), jnp.int32),
            pltpu.VMEM((ROWS_PER_W,), jnp.int32),
            pltpu.VMEM((CHUNK * MODEL_DIM,), jnp.float32),
            pltpu.VMEM((CHUNK * MODEL_DIM,), jnp.float32),
            pltpu.SemaphoreType.DMA((NBUF,)),
        ],
    )
    return run(pt_i32, pw_i32, tflat, wflat)


@jax.jit
def kernel(time, weekday, time_table, W_time, weekday_table, W_weekday):
    B, T, N = time.shape
    time2d = time.reshape(B * T, N).astype(jnp.int32)
    wd2d = weekday.reshape(B * T, 1).astype(jnp.int32)
    pt_i32, pw_i32, tflat, wflat = _build_tables(
        time2d, wd2d, time_table, W_time, weekday_table, W_weekday)
    out = _sc_gather(pt_i32, pw_i32, tflat, wflat)
    return out.reshape(B, T, N, MODEL_DIM)


# speedup vs baseline: 9.4174x; 1.7826x over previous
import functools

import jax
import jax.numpy as jnp
from jax import lax
from jax.experimental import pallas as pl
from jax.experimental.pallas import tpu as pltpu
from jax.experimental.pallas import tpu_sc as plsc

NUM_TIMES = 288
NUM_WEEKDAYS = 7
TIME_DIM = 64
WEEKDAY_DIM = 64
MODEL_DIM = 512

NC = 2
NS = 16
NW = NC * NS

TOKENS = 16 * 12 * 512
ROWS_PER_W = TOKENS // NW
CHUNK = 64
NBUF = 3
NCHUNK = ROWS_PER_W // CHUNK


def _tables_body(time_ref, wd_ref, tt_ref, wt_ref, wdt_ref, ww_ref,
                 c_ref, idx_ref):
    p_time = jnp.dot(tt_ref[...], wt_ref[...],
                     preferred_element_type=jnp.float32)
    p_wd = jnp.dot(wdt_ref[...], ww_ref[...],
                   preferred_element_type=jnp.float32)
    c_ref[...] = p_time[:, None, :] + p_wd[None, :, :]
    idx_ref[...] = time_ref[...] * NUM_WEEKDAYS + wd_ref[...]


def _build_tables(time2d, wd2d, time_table, W_time, weekday_table, W_weekday):
    c3, idx = pl.pallas_call(
        _tables_body,
        out_shape=[
            jax.ShapeDtypeStruct((NUM_TIMES, NUM_WEEKDAYS, MODEL_DIM),
                                 jnp.float32),
            jax.ShapeDtypeStruct(time2d.shape, jnp.int32),
        ],
    )(time2d, wd2d, time_table, W_time, weekday_table, W_weekday)
    return c3.reshape(NUM_TIMES * NUM_WEEKDAYS, MODEL_DIM), idx.reshape(-1)


def _gather_body(c_hbm, idx_hbm, out_hbm, idx_v, bufs, gsems, wsems):
    wid = lax.axis_index("s") * NC + lax.axis_index("c")
    base = wid * ROWS_PER_W
    pltpu.sync_copy(idx_hbm.at[pl.ds(base, ROWS_PER_W)], idx_v)

    def gather(c, b):
        pltpu.async_copy(
            c_hbm.at[idx_v.at[pl.ds(c * CHUNK, CHUNK)]], bufs.at[b],
            gsems.at[b])

    def write(c, b):
        pltpu.async_copy(
            bufs.at[b], out_hbm.at[pl.ds(base + c * CHUNK, CHUNK)],
            wsems.at[b])

    def wait_gather(b):
        pltpu.make_async_copy(c_hbm.at[pl.ds(0, CHUNK)], bufs.at[b],
                              gsems.at[b]).wait()

    def wait_write(b):
        pltpu.make_async_copy(bufs.at[b], out_hbm.at[pl.ds(0, CHUNK)],
                              wsems.at[b]).wait()

    for b in range(NBUF):
        gather(b, b)

    def body(g):
        for b in range(NBUF):
            c = g + b
            wait_gather(b)
            write(c, b)
        for b in range(NBUF):
            nc = g + NBUF + b

            @pl.when(nc < NCHUNK)
            def _():
                wait_write(b)
                gather(nc, b)

    pl.loop(0, NCHUNK, step=NBUF)(body)
    for b in range(NBUF):
        wait_write(b)


def _sc_gather(combined, idx):
    mesh = plsc.VectorSubcoreMesh(core_axis_name="c", subcore_axis_name="s")
    run = pl.kernel(
        _gather_body,
        out_type=jax.ShapeDtypeStruct((TOKENS, MODEL_DIM), jnp.float32),
        mesh=mesh,
        scratch_types=[
            pltpu.VMEM((ROWS_PER_W,), jnp.int32),
            pltpu.VMEM((NBUF, CHUNK, MODEL_DIM), jnp.float32),
            pltpu.SemaphoreType.DMA((NBUF,)),
            pltpu.SemaphoreType.DMA((NBUF,)),
        ],
    )
    return run(combined, idx)


@jax.jit
def kernel(time, weekday, time_table, W_time, weekday_table, W_weekday):
    B, T, N = time.shape
    time2d = time.reshape(B * T, N).astype(jnp.int32)
    wd2d = weekday.reshape(B * T, 1).astype(jnp.int32)
    combined, idx = _build_tables(time2d, wd2d, time_table, W_time,
                                  weekday_table, W_weekday)
    out = _sc_gather(combined, idx)
    return out.reshape(B, T, N, MODEL_DIM)
